# baseline (speedup 1.0000x reference)
import functools

import jax
import jax.numpy as jnp
from jax import lax
from jax.experimental import pallas as pl
from jax.experimental.pallas import tpu as pltpu
from jax.experimental.pallas import tpu_sc as plsc

I_SIZE = 128
O_SIZE = 64
BATCH = 1024
L = 401
LP = 408
Y_MEAN = 1.05
Y_RANGE = 1.9
SCALE = 10.0

PARTS = (32, 64, 32)
FPG = 4
N_WORKERS = 32
CHUNK = 128


def _tc_stage_a(zT_ref, w_ref, x_ref, table_ref, idx_ref, wx_ref):
    s = pl.program_id(0)

    wrow = w_ref[0, 0, :]
    fy = (2.0 * (Y_MEAN - jnp.abs(wrow)) / Y_RANGE + 1.0) * 0.5 * (L - 1)
    fy = jnp.clip(fy, 0.0, L - 1)
    y0f = jnp.floor(fy)
    wy = fy - y0f
    y0 = y0f.astype(jnp.int32)
    y1 = jnp.minimum(y0 + 1, L - 1)
    riota = lax.broadcasted_iota(jnp.int32, (LP, FPG * O_SIZE), 0)
    w2hot = jnp.where(riota == y0[None, :], (1.0 - wy)[None, :], 0.0)
    w2hot = w2hot + jnp.where(riota == y1[None, :], wy[None, :], 0.0)
    r = lax.dot_general(zT_ref[...], w2hot, (((1,), (0,)), ((), ())),
                        preferred_element_type=jnp.float32)
    r = r * SCALE
    blocks = []
    for j in range(FPG):
        t = r[:, j * O_SIZE:(j + 1) * O_SIZE]
        tsh = jnp.concatenate(
            [t[1:], jnp.zeros((1, O_SIZE), jnp.float32)], axis=0)
        blocks.append(jnp.concatenate([t, tsh - t], axis=1))
    table_ref[...] = jnp.concatenate(blocks, axis=0)

    xv = x_ref[0]
    fx = (2.0 - 2.0 * xv) * 0.5 * (L - 1)
    fx = jnp.clip(fx, 0.0, L - 1)
    x0f = jnp.floor(fx)
    feat = lax.broadcasted_iota(jnp.int32, (FPG, BATCH), 0) + FPG * s
    idx = feat * LP + x0f.astype(jnp.int32)
    idx_ref[...] = idx.reshape(FPG * BATCH // CHUNK, CHUNK)
    wx_ref[...] = (fx - x0f).reshape(FPG * BATCH // CHUNK, CHUNK)


def _make_sc_body(out_row0, n_chunks):
    rows_per_w = n_chunks * CHUNK

    def _sc_stage_b(table_hbm, idx_hbm, wx_hbm, out_hbm,
                    idx_v, wx_v, g0_v, g1_v, o0_v, o1_v,
                    sg0, sg1, sw0, sw1):
        wid = lax.axis_index("s") * 2 + lax.axis_index("c")
        irow0 = wid * n_chunks
        base = out_row0 + wid * rows_per_w

        pltpu.sync_copy(idx_hbm.at[pl.ds(irow0, n_chunks)], idx_v)
        pltpu.sync_copy(wx_hbm.at[pl.ds(irow0, n_chunks)], wx_v)

        g_bufs = (g0_v, g1_v)
        o_bufs = (o0_v, o1_v)
        g_sems = (sg0, sg1)
        w_sems = (sw0, sw1)

        def gather(c, slot):
            return pltpu.make_async_copy(
                table_hbm.at[idx_v.at[c]], g_bufs[slot], g_sems[slot])

        def writeback(c, slot):
            return pltpu.make_async_copy(
                o_bufs[slot], out_hbm.at[pl.ds(base + c * CHUNK, CHUNK)],
                w_sems[slot])

        gather(0, 0).start()
        gather(1, 1).start()

        def pair_body(p, carry):
            c0 = 2 * p
            for slot in range(2):
                c = c0 + slot
                g_buf, o_buf = g_bufs[slot], o_bufs[slot]
                gather(c, slot).wait()

                @pl.when(p > 0)
                def _():
                    writeback(c - 2, slot).wait()

                def group_body(g, carry2):
                    wvec = wx_v[c, pl.ds(g * 16, 16)]
                    for j in range(16):
                        row = g * 16 + j
                        wxs = wvec[j]
                        for k in range(O_SIZE // 16):
                            tv = g_buf[row, pl.ds(16 * k, 16)]
                            dv = g_buf[row, pl.ds(O_SIZE + 16 * k, 16)]
                            o_buf[row, pl.ds(16 * k, 16)] = tv + wxs * dv
                    return carry2

                lax.fori_loop(0, CHUNK // 16, group_body, 0)
                writeback(c, slot).start()

                @pl.when(c + 2 < n_chunks)
                def _():
                    gather(c + 2, slot).start()
            return carry

        lax.fori_loop(0, n_chunks // 2, pair_body, 0)
        writeback(n_chunks - 2, 0).wait()
        writeback(n_chunks - 1, 1).wait()

    return _sc_stage_b


def _sc_scratch(n_chunks):
    return [
        pltpu.VMEM((n_chunks, CHUNK), jnp.int32),
        pltpu.VMEM((n_chunks, CHUNK), jnp.float32),
        pltpu.VMEM((CHUNK, 2 * O_SIZE), jnp.float32),
        pltpu.VMEM((CHUNK, 2 * O_SIZE), jnp.float32),
        pltpu.VMEM((CHUNK, O_SIZE), jnp.float32),
        pltpu.VMEM((CHUNK, O_SIZE), jnp.float32),
        pltpu.SemaphoreType.DMA,
        pltpu.SemaphoreType.DMA,
        pltpu.SemaphoreType.DMA,
        pltpu.SemaphoreType.DMA,
    ]


def _tc_part(zT_pad, w3, xT3, ih):
    n_steps = ih // FPG
    return pl.pallas_call(
        _tc_stage_a,
        grid=(n_steps,),
        in_specs=[
            pl.BlockSpec((LP, LP), lambda s: (0, 0)),
            pl.BlockSpec((1, 1, FPG * O_SIZE), lambda s: (s, 0, 0)),
            pl.BlockSpec((1, FPG, BATCH), lambda s: (s, 0, 0)),
        ],
        out_specs=[
            pl.BlockSpec((FPG * LP, 2 * O_SIZE), lambda s: (s, 0)),
            pl.BlockSpec((FPG * BATCH // CHUNK, CHUNK), lambda s: (s, 0)),
            pl.BlockSpec((FPG * BATCH // CHUNK, CHUNK), lambda s: (s, 0)),
        ],
        out_shape=[
            jax.ShapeDtypeStruct((ih * LP, 2 * O_SIZE), jnp.float32),
            jax.ShapeDtypeStruct((ih * BATCH // CHUNK, CHUNK), jnp.int32),
            jax.ShapeDtypeStruct((ih * BATCH // CHUNK, CHUNK), jnp.float32),
        ],
    )(zT_pad, w3, xT3)


def kernel(weight, x, x_table, z_table):
    del x_table
    zT_pad = jnp.pad(jnp.transpose(z_table), ((0, LP - L), (0, LP - L)))
    xT = jnp.transpose(x)

    mesh = plsc.VectorSubcoreMesh(core_axis_name="c", subcore_axis_name="s")

    tc_outs = []
    f0 = 0
    for ih in PARTS:
        w3 = weight[f0:f0 + ih].reshape(ih // FPG, 1, FPG * O_SIZE)
        x3 = xT[f0:f0 + ih].reshape(ih // FPG, FPG, BATCH)
        tc_outs.append(_tc_part(zT_pad, w3, x3, ih))
        f0 += ih

    ref = None
    f0 = 0
    for p, ih in enumerate(PARTS):
        nc = ih * BATCH // (N_WORKERS * CHUNK)
        table, idx2d, wx2d = tc_outs[p]
        if p == 0:
            sc = functools.partial(
                pl.kernel,
                mesh=mesh,
                out_type=jax.ShapeDtypeStruct(
                    (I_SIZE * BATCH, O_SIZE), jnp.float32),
                scratch_types=_sc_scratch(nc),
            )(_make_sc_body(0, nc))
            ref = jax.new_ref(sc(table, idx2d, wx2d))
        else:
            sc = functools.partial(
                pl.kernel,
                mesh=mesh,
                out_type=(),
                scratch_types=_sc_scratch(nc),
            )(_make_sc_body(f0 * BATCH, nc))
            sc(table, idx2d, wx2d, ref)
        f0 += ih
    out = ref[...]
    return out.reshape(I_SIZE, BATCH, O_SIZE)


# speedup vs baseline: 1.0417x; 1.0417x over previous
import functools

import jax
import jax.numpy as jnp
from jax import lax
from jax.experimental import pallas as pl
from jax.experimental.pallas import tpu as pltpu
from jax.experimental.pallas import tpu_sc as plsc

I_SIZE = 128
O_SIZE = 64
BATCH = 1024
L = 401
LP = 408
Y_MEAN = 1.05
Y_RANGE = 1.9
SCALE = 10.0

I1 = 32
I2 = I_SIZE - I1
FPG = 4
N_WORKERS = 32
CHUNK = 128


def _tc_stage_a(zT_ref, w_ref, x_ref, table_ref, idx_ref, wx_ref):
    s = pl.program_id(0)

    wrow = w_ref[0, 0, :]
    fy = (2.0 * (Y_MEAN - jnp.abs(wrow)) / Y_RANGE + 1.0) * 0.5 * (L - 1)
    fy = jnp.clip(fy, 0.0, L - 1)
    y0f = jnp.floor(fy)
    wy = fy - y0f
    y0 = y0f.astype(jnp.int32)
    y1 = jnp.minimum(y0 + 1, L - 1)
    riota = lax.broadcasted_iota(jnp.int32, (LP, FPG * O_SIZE), 0)
    w2hot = jnp.where(riota == y0[None, :], (1.0 - wy)[None, :], 0.0)
    w2hot = w2hot + jnp.where(riota == y1[None, :], wy[None, :], 0.0)
    r = lax.dot_general(zT_ref[...], w2hot, (((1,), (0,)), ((), ())),
                        preferred_element_type=jnp.float32)
    r = r * SCALE
    blocks = []
    for j in range(FPG):
        t = r[:, j * O_SIZE:(j + 1) * O_SIZE]
        tsh = jnp.concatenate(
            [t[1:], jnp.zeros((1, O_SIZE), jnp.float32)], axis=0)
        blocks.append(jnp.concatenate([t, tsh - t], axis=1))
    table_ref[...] = jnp.concatenate(blocks, axis=0)

    xv = x_ref[0]
    fx = (2.0 - 2.0 * xv) * 0.5 * (L - 1)
    fx = jnp.clip(fx, 0.0, L - 1)
    x0f = jnp.floor(fx)
    feat = lax.broadcasted_iota(jnp.int32, (FPG, BATCH), 0) + FPG * s
    idx = feat * LP + x0f.astype(jnp.int32)
    idx_ref[...] = idx.reshape(FPG * BATCH // CHUNK, CHUNK)
    wx_ref[...] = (fx - x0f).reshape(FPG * BATCH // CHUNK, CHUNK)


def _make_sc_body(out_row0, n_chunks):
    rows_per_w = n_chunks * CHUNK

    def _sc_stage_b(table_hbm, idx_hbm, wx_hbm, out_hbm,
                    idx_v, wx_v, g0_v, g1_v, o0_v, o1_v,
                    sg0, sg1, sw0, sw1):
        wid = lax.axis_index("s") * 2 + lax.axis_index("c")
        irow0 = wid * n_chunks
        base = out_row0 + wid * rows_per_w

        pltpu.sync_copy(idx_hbm.at[pl.ds(irow0, n_chunks)], idx_v)
        pltpu.sync_copy(wx_hbm.at[pl.ds(irow0, n_chunks)], wx_v)

        g_bufs = (g0_v, g1_v)
        o_bufs = (o0_v, o1_v)
        g_sems = (sg0, sg1)
        w_sems = (sw0, sw1)

        def gather(c, slot):
            return pltpu.make_async_copy(
                table_hbm.at[idx_v.at[c]], g_bufs[slot], g_sems[slot])

        def writeback(c, slot):
            return pltpu.make_async_copy(
                o_bufs[slot], out_hbm.at[pl.ds(base + c * CHUNK, CHUNK)],
                w_sems[slot])

        gather(0, 0).start()
        gather(1, 1).start()

        def pair_body(p, carry):
            c0 = 2 * p
            for slot in range(2):
                c = c0 + slot
                g_buf, o_buf = g_bufs[slot], o_bufs[slot]
                gather(c, slot).wait()

                @pl.when(p > 0)
                def _():
                    writeback(c - 2, slot).wait()

                def group_body(g, carry2):
                    wvec = wx_v[c, pl.ds(g * 16, 16)]
                    for j in range(16):
                        row = g * 16 + j
                        wxs = wvec[j]
                        for k in range(O_SIZE // 16):
                            tv = g_buf[row, pl.ds(16 * k, 16)]
                            dv = g_buf[row, pl.ds(O_SIZE + 16 * k, 16)]
                            o_buf[row, pl.ds(16 * k, 16)] = tv + wxs * dv
                    return carry2

                lax.fori_loop(0, CHUNK // 16, group_body, 0)
                writeback(c, slot).start()

                @pl.when(c + 2 < n_chunks)
                def _():
                    gather(c + 2, slot).start()
            return carry

        lax.fori_loop(0, n_chunks // 2, pair_body, 0)
        writeback(n_chunks - 2, 0).wait()
        writeback(n_chunks - 1, 1).wait()

    return _sc_stage_b


def _sc_scratch(n_chunks):
    return [
        pltpu.VMEM((n_chunks, CHUNK), jnp.int32),
        pltpu.VMEM((n_chunks, CHUNK), jnp.float32),
        pltpu.VMEM((CHUNK, 2 * O_SIZE), jnp.float32),
        pltpu.VMEM((CHUNK, 2 * O_SIZE), jnp.float32),
        pltpu.VMEM((CHUNK, O_SIZE), jnp.float32),
        pltpu.VMEM((CHUNK, O_SIZE), jnp.float32),
        pltpu.SemaphoreType.DMA,
        pltpu.SemaphoreType.DMA,
        pltpu.SemaphoreType.DMA,
        pltpu.SemaphoreType.DMA,
    ]


def _tc_part(zT_pad, w3, xT3, ih):
    n_steps = ih // FPG
    return pl.pallas_call(
        _tc_stage_a,
        grid=(n_steps,),
        in_specs=[
            pl.BlockSpec((LP, LP), lambda s: (0, 0)),
            pl.BlockSpec((1, 1, FPG * O_SIZE), lambda s: (s, 0, 0)),
            pl.BlockSpec((1, FPG, BATCH), lambda s: (s, 0, 0)),
        ],
        out_specs=[
            pl.BlockSpec((FPG * LP, 2 * O_SIZE), lambda s: (s, 0)),
            pl.BlockSpec((FPG * BATCH // CHUNK, CHUNK), lambda s: (s, 0)),
            pl.BlockSpec((FPG * BATCH // CHUNK, CHUNK), lambda s: (s, 0)),
        ],
        out_shape=[
            jax.ShapeDtypeStruct((ih * LP, 2 * O_SIZE), jnp.float32),
            jax.ShapeDtypeStruct((ih * BATCH // CHUNK, CHUNK), jnp.int32),
            jax.ShapeDtypeStruct((ih * BATCH // CHUNK, CHUNK), jnp.float32),
        ],
    )(zT_pad, w3, xT3)


def kernel(weight, x, x_table, z_table):
    del x_table
    zT_pad = jnp.pad(jnp.transpose(z_table), ((0, LP - L), (0, LP - L)))
    xT = jnp.transpose(x)
    w3a = weight[:I1].reshape(I1 // FPG, 1, FPG * O_SIZE)
    w3b = weight[I1:].reshape(I2 // FPG, 1, FPG * O_SIZE)
    xTa = xT[:I1].reshape(I1 // FPG, FPG, BATCH)
    xTb = xT[I1:].reshape(I2 // FPG, FPG, BATCH)

    mesh = plsc.VectorSubcoreMesh(core_axis_name="c", subcore_axis_name="s")

    table0, idx0, wx0 = _tc_part(zT_pad, w3a, xTa, I1)
    table1, idx1, wx1 = _tc_part(zT_pad, w3b, xTb, I2)

    nc1 = I1 * BATCH // (N_WORKERS * CHUNK)
    nc2 = I2 * BATCH // (N_WORKERS * CHUNK)
    sc0 = functools.partial(
        pl.kernel,
        mesh=mesh,
        out_type=jax.ShapeDtypeStruct((I_SIZE * BATCH, O_SIZE), jnp.float32),
        scratch_types=_sc_scratch(nc1),
    )(_make_sc_body(0, nc1))
    out0 = sc0(table0, idx0, wx0)

    ref = jax.new_ref(out0)
    sc1 = functools.partial(
        pl.kernel,
        mesh=mesh,
        out_type=(),
        scratch_types=_sc_scratch(nc2),
    )(_make_sc_body(I1 * BATCH, nc2))
    sc1(table1, idx1, wx1, ref)
    out = ref[...]
    return out.reshape(I_SIZE, BATCH, O_SIZE)
